# Initial kernel scaffold; baseline (speedup 1.0000x reference)
import jax
import jax.numpy as jnp
from jax.experimental import pallas as pl


def kernel(X, z, r):
    raise NotImplementedError("write your pallas kernel here")


# speedup vs baseline: 21.0131x; 21.0131x over previous
import jax
import jax.numpy as jnp
from jax.experimental import pallas as pl
from jax.experimental.pallas import tpu as pltpu

K = 32
EPS = 1e-6
BLK = 512


def _stats_kernel(z_ref, x_ref, r_ref, ns_ref, mus_ref, covs_ref, *, nblocks, s):
    step = pl.program_id(0)

    @pl.when(step == 0)
    def _init():
        ns_ref[...] = jnp.zeros_like(ns_ref)
        mus_ref[...] = jnp.zeros_like(mus_ref)
        covs_ref[...] = jnp.zeros_like(covs_ref)

    x = x_ref[...]
    r = r_ref[...]
    zc = z_ref[...]
    zmin = zc[0, 0]
    zmax = zc[BLK - 1, 0]

    def seg_body(k, carry):
        mask = zc == k
        for j in range(s):
            w = jnp.where(mask, r[:, j:j + 1], 0.0)
            xw = x * w
            y = jax.lax.dot_general(
                xw, x, (((0,), (0,)), ((), ())),
                preferred_element_type=jnp.float32)
            g = k * s + j
            covs_ref[pl.ds(g, 1)] += y[None]
            mus_ref[pl.ds(g, 1)] += jnp.sum(xw, axis=0, keepdims=True)
            ns_ref[pl.ds(g, 1)] += jnp.sum(w, axis=0, keepdims=True)
        return carry

    jax.lax.fori_loop(zmin, zmax + 1, seg_body, 0)

    @pl.when(step == nblocks - 1)
    def _epilogue():
        denom = ns_ref[...] + EPS
        mus = mus_ref[...] / denom
        mus_ref[...] = mus
        covs_ref[...] = (covs_ref[...] / denom[:, :, None]
                         - mus[:, :, None] * mus[:, None, :])


def kernel(X, z, r):
    n, d = X.shape
    s = r.shape[1]
    g = K * s
    nblocks = n // BLK
    z2 = z.astype(jnp.int32).reshape(n, 1)

    ns, mus, covs = pl.pallas_call(
        lambda *refs: _stats_kernel(*refs, nblocks=nblocks, s=s),
        grid=(nblocks,),
        in_specs=[
            pl.BlockSpec((BLK, 1), lambda i: (i, 0)),
            pl.BlockSpec((BLK, d), lambda i: (i, 0)),
            pl.BlockSpec((BLK, s), lambda i: (i, 0)),
        ],
        out_specs=[
            pl.BlockSpec((g, 1), lambda i: (0, 0)),
            pl.BlockSpec((g, d), lambda i: (0, 0)),
            pl.BlockSpec((g, d, d), lambda i: (0, 0, 0)),
        ],
        out_shape=[
            jax.ShapeDtypeStruct((g, 1), jnp.float32),
            jax.ShapeDtypeStruct((g, d), jnp.float32),
            jax.ShapeDtypeStruct((g, d, d), jnp.float32),
        ],
        compiler_params=pltpu.CompilerParams(
            dimension_semantics=("arbitrary",)),
    )(z2, X, r)

    return ns.reshape(g), mus, covs
